# baseline (speedup 1.0000x reference)
import jax
import jax.numpy as jnp
from jax import lax
from jax.experimental import pallas as pl
from jax.experimental.pallas import tpu as pltpu, tpu_sc as plsc

_N_FIELDS = 26
_VOCAB = 100000
_V_PAD = 102400
_EMB_DIM = 16
_BATCH = 16384

_NC = 2
_NS = 16
_NW = _NC * _NS

_B_PER_W = _BATCH // _NW
_IDX_MINOR = 128
_SUBS = _B_PER_W // _IDX_MINOR
_GATHER_BATCH = 13

_V_CHUNK = 10240
_N_VC = _V_PAD // _V_CHUNK


def _tc_reduce_body(t_ref, o_ref):
    o_ref[...] = jnp.sum(t_ref[0], axis=0)


def _sc_body(s_hbm, x_hbm, w_hbm, out_hbm, idx_v, g_v, w_v, out_v, sem):
    wid = lax.axis_index("s") * _NC + lax.axis_index("c")

    pltpu.sync_copy(x_hbm.at[:, pl.ds(wid * _SUBS, _SUBS), :], idx_v)
    pltpu.sync_copy(w_hbm, w_v)
    w_regs = [w_v[f] for f in range(_N_FIELDS)]

    total = _N_FIELDS * _SUBS
    for batch in range(pl.cdiv(total, _GATHER_BATCH)):
        copies = []
        for r in range(_GATHER_BATCH):
            row = batch * _GATHER_BATCH + r
            if row >= total:
                break
            f, s = divmod(row, _SUBS)
            copies.append(pltpu.async_copy(
                s_hbm.at[f].at[idx_v.at[f, s]],
                g_v.at[f, pl.ds(s * _IDX_MINOR, _IDX_MINOR)],
                sem))
        for cp in copies:
            cp.wait()

    def group(g16, carry):
        b0 = g16 * 16
        acc = g_v[0, pl.ds(b0, 16)] * w_regs[0]
        for f in range(1, _N_FIELDS):
            acc = acc + g_v[f, pl.ds(b0, 16)] * w_regs[f]
        out_v[pl.ds(b0, 16)] = acc
        return carry

    lax.fori_loop(0, _B_PER_W // 16, group, 0)

    pltpu.sync_copy(out_v, out_hbm.at[pl.ds(wid * _B_PER_W, _B_PER_W)])


@jax.jit
def _run(tbl_t, xt, wmat):
    s_flat = pl.pallas_call(
        _tc_reduce_body,
        grid=(_N_FIELDS, _N_VC),
        in_specs=[pl.BlockSpec((1, _EMB_DIM, _V_CHUNK), lambda f, c: (f, 0, c))],
        out_specs=pl.BlockSpec((_V_CHUNK,), lambda f, c: (f * _N_VC + c,)),
        out_shape=jax.ShapeDtypeStruct((_N_FIELDS * _V_PAD,), jnp.float32),
    )(tbl_t)
    s2 = s_flat.reshape(_N_FIELDS, _V_PAD)

    mesh = plsc.VectorSubcoreMesh(core_axis_name="c", subcore_axis_name="s")
    f = pl.kernel(
        _sc_body,
        mesh=mesh,
        compiler_params=pltpu.CompilerParams(
            needs_layout_passes=False, use_tc_tiling_on_sc=False),
        out_type=jax.ShapeDtypeStruct((_BATCH,), jnp.float32),
        scratch_types=[
            pltpu.VMEM((_N_FIELDS, _SUBS, _IDX_MINOR), jnp.int32),
            pltpu.VMEM((_N_FIELDS, _B_PER_W), jnp.float32),
            pltpu.VMEM((_N_FIELDS, _EMB_DIM), jnp.float32),
            pltpu.VMEM((_B_PER_W,), jnp.float32),
            pltpu.SemaphoreType.DMA,
        ],
    )
    return f(s2, xt, wmat)


def kernel(X, tables, alpha):
    w = jnp.tanh(alpha).astype(jnp.float32)
    wmat = jnp.broadcast_to(w[:, None], (_N_FIELDS, _EMB_DIM))
    tbl_t = jnp.transpose(tables, (0, 2, 1))
    xt = jnp.transpose(X, (1, 0)).reshape(-1).reshape(
        _N_FIELDS, _BATCH // _IDX_MINOR, _IDX_MINOR)
    out = _run(tbl_t, xt, wmat)
    return out[:, None]


# speedup vs baseline: 2.1050x; 2.1050x over previous
import jax
import jax.numpy as jnp
from jax import lax
from jax.experimental import pallas as pl
from jax.experimental.pallas import tpu as pltpu, tpu_sc as plsc

_N_FIELDS = 26
_VOCAB = 100000
_V_PAD = 102400
_EMB_DIM = 16
_BATCH = 16384

_NC = 2
_NS = 16
_NW = _NC * _NS

_B_PER_W = _BATCH // _NW
_IDX_MINOR = 128
_SUBS = _B_PER_W // _IDX_MINOR
_GATHER_BATCH = 13

_V_CHUNK = _V_PAD
_N_VC = _V_PAD // _V_CHUNK


def _tc_reduce_body(t_ref, o_ref):
    o_ref[...] = jnp.sum(t_ref[0], axis=0)


def _sc_body(s_hbm, x_hbm, w_hbm, out_hbm, idx_v, g_v, w_v, out_v, sem):
    wid = lax.axis_index("s") * _NC + lax.axis_index("c")

    pltpu.sync_copy(x_hbm.at[:, pl.ds(wid * _SUBS, _SUBS), :], idx_v)
    pltpu.sync_copy(w_hbm, w_v)
    w_regs = [w_v[f] for f in range(_N_FIELDS)]

    total = _N_FIELDS * _SUBS
    for batch in range(pl.cdiv(total, _GATHER_BATCH)):
        copies = []
        for r in range(_GATHER_BATCH):
            row = batch * _GATHER_BATCH + r
            if row >= total:
                break
            f, s = divmod(row, _SUBS)
            copies.append(pltpu.async_copy(
                s_hbm.at[f].at[idx_v.at[f, s]],
                g_v.at[f, pl.ds(s * _IDX_MINOR, _IDX_MINOR)],
                sem))
        for cp in copies:
            cp.wait()

    def group(g16, carry):
        b0 = g16 * 16
        acc = g_v[0, pl.ds(b0, 16)] * w_regs[0]
        for f in range(1, _N_FIELDS):
            acc = acc + g_v[f, pl.ds(b0, 16)] * w_regs[f]
        out_v[pl.ds(b0, 16)] = acc
        return carry

    lax.fori_loop(0, _B_PER_W // 16, group, 0)

    pltpu.sync_copy(out_v, out_hbm.at[pl.ds(wid * _B_PER_W, _B_PER_W)])


@jax.jit
def _run(tbl_t, xt, wmat):
    s_flat = pl.pallas_call(
        _tc_reduce_body,
        grid=(_N_FIELDS, _N_VC),
        in_specs=[pl.BlockSpec((1, _EMB_DIM, _V_CHUNK), lambda f, c: (f, 0, c))],
        out_specs=pl.BlockSpec((_V_CHUNK,), lambda f, c: (f * _N_VC + c,)),
        out_shape=jax.ShapeDtypeStruct((_N_FIELDS * _V_PAD,), jnp.float32),
    )(tbl_t)
    s2 = s_flat.reshape(_N_FIELDS, _V_PAD)

    mesh = plsc.VectorSubcoreMesh(core_axis_name="c", subcore_axis_name="s")
    f = pl.kernel(
        _sc_body,
        mesh=mesh,
        compiler_params=pltpu.CompilerParams(
            needs_layout_passes=False, use_tc_tiling_on_sc=False),
        out_type=jax.ShapeDtypeStruct((_BATCH,), jnp.float32),
        scratch_types=[
            pltpu.VMEM((_N_FIELDS, _SUBS, _IDX_MINOR), jnp.int32),
            pltpu.VMEM((_N_FIELDS, _B_PER_W), jnp.float32),
            pltpu.VMEM((_N_FIELDS, _EMB_DIM), jnp.float32),
            pltpu.VMEM((_B_PER_W,), jnp.float32),
            pltpu.SemaphoreType.DMA,
        ],
    )
    return f(s2, xt, wmat)


def kernel(X, tables, alpha):
    w = jnp.tanh(alpha).astype(jnp.float32)
    wmat = jnp.broadcast_to(w[:, None], (_N_FIELDS, _EMB_DIM))
    tbl_t = jnp.transpose(tables, (0, 2, 1))
    xt = jnp.transpose(X, (1, 0)).reshape(-1).reshape(
        _N_FIELDS, _BATCH // _IDX_MINOR, _IDX_MINOR)
    out = _run(tbl_t, xt, wmat)
    return out[:, None]


# speedup vs baseline: 2.1222x; 1.0082x over previous
import jax
import jax.numpy as jnp
from jax import lax
from jax.experimental import pallas as pl
from jax.experimental.pallas import tpu as pltpu, tpu_sc as plsc

_N_FIELDS = 26
_VOCAB = 100000
_V_PAD = 102400
_EMB_DIM = 16
_BATCH = 16384

_NC = 2
_NS = 16
_NW = _NC * _NS

_B_PER_W = _BATCH // _NW
_IDX_MINOR = 128
_SUBS = _B_PER_W // _IDX_MINOR
_GATHER_BATCH = 13

_V_CHUNK = _V_PAD
_F_CHUNK = 2


def _tc_reduce_body(t_ref, o_ref):
    for i in range(_F_CHUNK):
        o_ref[pl.ds(i * _V_PAD, _V_PAD)] = jnp.sum(t_ref[i], axis=0)


def _sc_body(s_hbm, x_hbm, w_hbm, out_hbm, idx_v, g_v, w_v, out_v, sem):
    wid = lax.axis_index("s") * _NC + lax.axis_index("c")

    pltpu.sync_copy(x_hbm.at[:, pl.ds(wid * _SUBS, _SUBS), :], idx_v)
    pltpu.sync_copy(w_hbm, w_v)
    w_regs = [w_v[f] for f in range(_N_FIELDS)]

    total = _N_FIELDS * _SUBS
    for batch in range(pl.cdiv(total, _GATHER_BATCH)):
        copies = []
        for r in range(_GATHER_BATCH):
            row = batch * _GATHER_BATCH + r
            if row >= total:
                break
            f, s = divmod(row, _SUBS)
            copies.append(pltpu.async_copy(
                s_hbm.at[f].at[idx_v.at[f, s]],
                g_v.at[f, pl.ds(s * _IDX_MINOR, _IDX_MINOR)],
                sem))
        for cp in copies:
            cp.wait()

    def group(g16, carry):
        b0 = g16 * 16
        acc = g_v[0, pl.ds(b0, 16)] * w_regs[0]
        for f in range(1, _N_FIELDS):
            acc = acc + g_v[f, pl.ds(b0, 16)] * w_regs[f]
        out_v[pl.ds(b0, 16)] = acc
        return carry

    lax.fori_loop(0, _B_PER_W // 16, group, 0)

    pltpu.sync_copy(out_v, out_hbm.at[pl.ds(wid * _B_PER_W, _B_PER_W)])


@jax.jit
def _run(tbl_t, xt, wmat):
    s_flat = pl.pallas_call(
        _tc_reduce_body,
        grid=(_N_FIELDS // _F_CHUNK,),
        in_specs=[pl.BlockSpec((_F_CHUNK, _EMB_DIM, _V_CHUNK),
                               lambda f: (f, 0, 0))],
        out_specs=pl.BlockSpec((_F_CHUNK * _V_PAD,), lambda f: (f,)),
        out_shape=jax.ShapeDtypeStruct((_N_FIELDS * _V_PAD,), jnp.float32),
    )(tbl_t)
    s2 = s_flat.reshape(_N_FIELDS, _V_PAD)

    mesh = plsc.VectorSubcoreMesh(core_axis_name="c", subcore_axis_name="s")
    f = pl.kernel(
        _sc_body,
        mesh=mesh,
        compiler_params=pltpu.CompilerParams(
            needs_layout_passes=False, use_tc_tiling_on_sc=False),
        out_type=jax.ShapeDtypeStruct((_BATCH,), jnp.float32),
        scratch_types=[
            pltpu.VMEM((_N_FIELDS, _SUBS, _IDX_MINOR), jnp.int32),
            pltpu.VMEM((_N_FIELDS, _B_PER_W), jnp.float32),
            pltpu.VMEM((_N_FIELDS, _EMB_DIM), jnp.float32),
            pltpu.VMEM((_B_PER_W,), jnp.float32),
            pltpu.SemaphoreType.DMA,
        ],
    )
    return f(s2, xt, wmat)


def kernel(X, tables, alpha):
    w = jnp.tanh(alpha).astype(jnp.float32)
    wmat = jnp.broadcast_to(w[:, None], (_N_FIELDS, _EMB_DIM))
    tbl_t = jnp.transpose(tables, (0, 2, 1))
    xt = jnp.transpose(X, (1, 0)).reshape(-1).reshape(
        _N_FIELDS, _BATCH // _IDX_MINOR, _IDX_MINOR)
    out = _run(tbl_t, xt, wmat)
    return out[:, None]
